# baseline (speedup 1.0000x reference)
import jax
import jax.numpy as jnp
from jax import lax
from jax.experimental import pallas as pl
from jax.experimental.pallas import tpu as pltpu
from jax.experimental.pallas import tpu_sc as plsc

N = 50000
KNB = 8
NC = 2
NS = 16
NW = NC * NS
BPW = 1568
VPW = BPW // 16


def _diff_body(ci, cp, fi, fp, *o_refs):
    for c in range(3):
        o_refs[c][...] = ci[c, :] - cp[c, :]
        o_refs[3 + c][...] = fi[c, :] - fp[c, :]


def _copy_idx_columns(idxF, idxv, g, cbase, sem):
    handles = [
        pltpu.async_copy(
            idxF.at[pl.ds((k * 2 + g) * N + cbase, BPW)],
            idxv.at[pl.ds(slot * BPW, BPW)], sem)
        for slot, k in enumerate(list(range(KNB)) + [KNB + 1])
    ]
    for h in handles:
        h.wait()


def _sc_body(d00, d01, d02, d10, d11, d12, idxF, out_hbm,
             table0, table1, idxv, invv, outv, sem0, sem1, semi):
    d_planes = (d00, d01, d02, d10, d11, d12)
    bufs = (table0, table1)
    sems = (sem0, sem1)
    wid = lax.axis_index("c") * NS + lax.axis_index("s")
    base = wid * BPW
    cbase = jnp.minimum(base, N - BPW)
    doff = base - cbase
    lossvec = jnp.zeros((16,), jnp.float32)

    pending = pltpu.async_copy(d_planes[0], bufs[0], sems[0])
    _copy_idx_columns(idxF, idxv, 0, cbase, semi)
    for i in range(6):
        g, c = divmod(i, 3)
        nxt = None
        if i + 1 < 6:
            nxt = pltpu.async_copy(
                d_planes[i + 1], bufs[(i + 1) % 2], sems[(i + 1) % 2])
        pending.wait()
        tbl = bufs[i % 2]

        def body(o, lv, tbl=tbl, first=(c == 0)):
            if first:
                deg = idxv[pl.ds(KNB * BPW + o, 16)]
                inv = 1.0 / deg.astype(jnp.float32)
                invv[pl.ds(o, 16)] = inv
            else:
                inv = invv[pl.ds(o, 16)]
            acc = jnp.zeros((16,), jnp.float32)
            for k in range(KNB):
                nb = idxv[pl.ds(k * BPW + o, 16)]
                acc = acc + plsc.load_gather(tbl, [nb])
            own = tbl[pl.ds(cbase + o, 16)]
            r = own - acc * inv
            return lv + r * r

        lossvec = plsc.parallel_loop(doff, BPW, 16, carry=lossvec)(body)
        if i == 2:
            _copy_idx_columns(idxF, idxv, 1, cbase, semi)
        pending = nxt
    outv[...] = lossvec
    pltpu.sync_copy(outv, out_hbm.at[pl.ds(wid * 16, 16)])


def kernel(coarse_input, coarse_pred, fine_input, fine_pred, laplace_idx_list):
    plane = jax.ShapeDtypeStruct((N,), jnp.float32)
    d_planes = pl.pallas_call(
        _diff_body,
        out_shape=[plane] * 6,
    )(coarse_input.T, coarse_pred.T, fine_input.T, fine_pred.T)

    idxF = jnp.transpose(laplace_idx_list, (2, 0, 1)).reshape(-1)

    mesh = plsc.VectorSubcoreMesh(core_axis_name="c", subcore_axis_name="s")
    part = pl.kernel(
        _sc_body,
        mesh=mesh,
        compiler_params=pltpu.CompilerParams(needs_layout_passes=False),
        out_type=jax.ShapeDtypeStruct((NW * 16,), jnp.float32),
        scratch_types=[
            pltpu.VMEM((N,), jnp.float32),
            pltpu.VMEM((N,), jnp.float32),
            pltpu.VMEM(((KNB + 1) * BPW,), jnp.int32),
            pltpu.VMEM((BPW,), jnp.float32),
            pltpu.VMEM((16,), jnp.float32),
            pltpu.SemaphoreType.DMA,
            pltpu.SemaphoreType.DMA,
            pltpu.SemaphoreType.DMA,
        ],
    )(*d_planes, idxF)
    return jnp.sum(part) * jnp.float32(0.5 / N)


# speedup vs baseline: 1.4021x; 1.4021x over previous
import jax
import jax.numpy as jnp
from jax import lax
from jax.experimental import pallas as pl
from jax.experimental.pallas import tpu as pltpu
from jax.experimental.pallas import tpu_sc as plsc

N = 50000
KNB = 8
NC = 2
NS = 16
NW = NC * NS
NPLANE = 6
NSUB = 5
SUB = N // NSUB
CH = 2000
NCHUNK = SUB // CH


def _diff_body(ci, cp, fi, fp, o_ref):
    for c in range(3):
        o_ref[pl.ds(c * N, N)] = ci[c, :] - cp[c, :]
        o_ref[pl.ds((3 + c) * N, N)] = fi[c, :] - fp[c, :]


def _fire_idx_chunk(idxF, ibuf, g, nstart, sem):
    return [
        pltpu.async_copy(
            idxF.at[pl.ds((k * 2 + g) * N + nstart, CH)],
            ibuf.at[pl.ds(slot * CH, CH)], sem)
        for slot, k in enumerate(list(range(KNB)) + [KNB + 1])
    ]


def _sc_body(d_flat, idxF, out_hbm, table, ibuf0, ibuf1, outv, semt, semi):
    ibufs = (ibuf0, ibuf1)
    wid = lax.axis_index("c") * NS + lax.axis_index("s")
    outv[...] = jnp.zeros((16,), jnp.float32)

    @pl.when(wid < NPLANE * NSUB)
    def _():
        p = wid % NPLANE
        sub = wid // NPLANE
        g = p // 3
        nbase = sub * SUB

        tcopy = pltpu.async_copy(d_flat.at[pl.ds(p * N, N)], table, semt)
        pend = _fire_idx_chunk(idxF, ibufs[0], g, nbase, semi)
        tcopy.wait()
        lossvec = jnp.zeros((16,), jnp.float32)
        for j in range(NCHUNK):
            nxt = None
            if j + 1 < NCHUNK:
                nxt = _fire_idx_chunk(
                    idxF, ibufs[(j + 1) % 2], g, nbase + (j + 1) * CH, semi)
            for h in pend:
                h.wait()
            ibuf = ibufs[j % 2]

            def body(o, lv, ibuf=ibuf, j=j):
                deg = ibuf[pl.ds(KNB * CH + o, 16)]
                inv = 1.0 / deg.astype(jnp.float32)
                acc = jnp.zeros((16,), jnp.float32)
                for k in range(KNB):
                    nb = ibuf[pl.ds(k * CH + o, 16)]
                    acc = acc + plsc.load_gather(table, [nb])
                own = table[pl.ds(nbase + j * CH + o, 16)]
                r = own - acc * inv
                return lv + r * r

            lossvec = plsc.parallel_loop(0, CH, 16, carry=lossvec)(body)
            pend = nxt
        outv[...] = lossvec

    pltpu.sync_copy(outv, out_hbm.at[pl.ds(wid * 16, 16)])


def kernel(coarse_input, coarse_pred, fine_input, fine_pred, laplace_idx_list):
    d_flat = pl.pallas_call(
        _diff_body,
        out_shape=jax.ShapeDtypeStruct((NPLANE * N,), jnp.float32),
    )(coarse_input.T, coarse_pred.T, fine_input.T, fine_pred.T)

    idxF = jnp.transpose(laplace_idx_list, (2, 0, 1)).reshape(-1)

    mesh = plsc.VectorSubcoreMesh(core_axis_name="c", subcore_axis_name="s")
    part = pl.kernel(
        _sc_body,
        mesh=mesh,
        compiler_params=pltpu.CompilerParams(needs_layout_passes=False),
        out_type=jax.ShapeDtypeStruct((NW * 16,), jnp.float32),
        scratch_types=[
            pltpu.VMEM((N,), jnp.float32),
            pltpu.VMEM(((KNB + 1) * CH,), jnp.int32),
            pltpu.VMEM(((KNB + 1) * CH,), jnp.int32),
            pltpu.VMEM((16,), jnp.float32),
            pltpu.SemaphoreType.DMA,
            pltpu.SemaphoreType.DMA,
        ],
    )(d_flat, idxF)
    return jnp.sum(part) * jnp.float32(0.5 / N)
